# Initial kernel scaffold; baseline (speedup 1.0000x reference)
import jax
import jax.numpy as jnp
from jax.experimental import pallas as pl


def kernel(x, edge_index, W_gcn, b_gcn, W1, b1, W2, b2):
    raise NotImplementedError("write your pallas kernel here")


# speedup vs baseline: 1.0657x; 1.0657x over previous
import functools

import jax
import jax.numpy as jnp
from jax import lax
from jax.experimental import pallas as pl
from jax.experimental.pallas import tpu as pltpu
from jax.experimental.pallas import tpu_sc as plsc

N, E, D, C = 10000, 320000, 128, 64
H = 4 * C

NC, NS, L = 2, 16, 16
NW = NC * NS
NPAD = 10240
RPT = NPAD // NS
KB = 64
EPW = 10112
NBATCH = EPW // KB
EPAD = EPW * NW

_mesh = plsc.VectorSubcoreMesh(
    core_axis_name="c", subcore_axis_name="s", num_cores=NC, num_subcores=NS)


def _tc1_body(x_ref, w_ref, o_ref):
    o_ref[...] = jnp.dot(x_ref[...], w_ref[...],
                         preferred_element_type=jnp.float32)


def _tc1_y(x, w_gcn):
    rb = 1000
    return pl.pallas_call(
        _tc1_body,
        grid=(N // rb,),
        in_specs=[
            pl.BlockSpec((rb, D), lambda i: (i, 0)),
            pl.BlockSpec((D, C), lambda i: (0, 0)),
        ],
        out_specs=pl.BlockSpec((rb, C), lambda i: (i, 0)),
        out_shape=jax.ShapeDtypeStruct((N, C), jnp.float32),
    )(x, w_gcn)


def _tc3_body(p_ref, bg_ref, w1a_ref, w1b_ref, b1_ref, u_ref, v_ref):
    emb = p_ref[0] + p_ref[1] + bg_ref[...]
    u_ref[...] = jnp.dot(emb, w1a_ref[...],
                         preferred_element_type=jnp.float32) + b1_ref[...]
    v_ref[...] = jnp.dot(emb, w1b_ref[...],
                         preferred_element_type=jnp.float32)


def _tc3_uv(partial, b_gcn, w1a, w1b, b1):
    rb = 1000
    return pl.pallas_call(
        _tc3_body,
        grid=(N // rb,),
        in_specs=[
            pl.BlockSpec((NC, rb, C), lambda i: (0, i, 0)),
            pl.BlockSpec((1, C), lambda i: (0, 0)),
            pl.BlockSpec((C, H), lambda i: (0, 0)),
            pl.BlockSpec((C, H), lambda i: (0, 0)),
            pl.BlockSpec((1, H), lambda i: (0, 0)),
        ],
        out_specs=[
            pl.BlockSpec((rb, H), lambda i: (i, 0)),
            pl.BlockSpec((rb, H), lambda i: (i, 0)),
        ],
        out_shape=[
            jax.ShapeDtypeStruct((N, H), jnp.float32),
            jax.ShapeDtypeStruct((N, H), jnp.float32),
        ],
    )(partial, b_gcn, w1a, w1b, b1)


def _tc5_body(q_ref, bg_ref, o_ref):
    o_ref[...] = q_ref[0] + q_ref[1] + bg_ref[...]


def _tc5_out(partial, b_gcn):
    rb = 1000
    return pl.pallas_call(
        _tc5_body,
        grid=(N // rb,),
        in_specs=[
            pl.BlockSpec((NC, rb, C), lambda i: (0, i, 0)),
            pl.BlockSpec((1, C), lambda i: (0, 0)),
        ],
        out_specs=pl.BlockSpec((rb, C), lambda i: (i, 0)),
        out_shape=jax.ShapeDtypeStruct((N, C), jnp.float32),
    )(partial, b_gcn)


ZR = 64


def _zero_accum_slice(zbuf, accum, roff):
    zvec = jnp.zeros((L,), jnp.float32)

    def zrow(r, _):
        for cc in range(C // L):
            zbuf[r, pl.ds(cc * L, L)] = zvec
        return _

    lax.fori_loop(0, ZR, zrow, None)
    for b in range(RPT // ZR):
        pltpu.sync_copy(zbuf, accum.at[pl.ds(roff + b * ZR, ZR)])


@functools.partial(
    pl.kernel,
    out_type=jax.ShapeDtypeStruct((NC, NPAD, C), jnp.float32),
    mesh=_mesh,
    compiler_params=pltpu.CompilerParams(use_tc_tiling_on_sc=False,
                                         needs_layout_passes=False),
    scratch_types=[
        pltpu.VMEM((NBATCH, KB), jnp.int32),
        pltpu.VMEM((NBATCH, KB), jnp.int32),
        pltpu.VMEM((KB, C), jnp.float32),
        pltpu.VMEM((ZR, C), jnp.float32),
        pltpu.VMEM_SHARED((NPAD, C), jnp.float32),
        pltpu.SemaphoreType.DMA,
    ],
)
def _sc2_segsum(y_hbm, src_hbm, dst_hbm, out_hbm,
                src_v, dst_v, rows_v, zbuf, accum, sem):
    cid = lax.axis_index("c")
    sid = lax.axis_index("s")
    wid = sid * NC + cid
    roff = pl.multiple_of(sid * RPT, 8)

    _zero_accum_slice(zbuf, accum, roff)
    pltpu.sync_copy(src_hbm.at[wid], src_v)
    pltpu.sync_copy(dst_hbm.at[wid], dst_v)
    plsc.subcore_barrier()

    def body(j, _):
        pltpu.async_copy(y_hbm.at[src_v.at[j]], rows_v, sem).wait()
        pltpu.sync_copy(rows_v, accum.at[dst_v.at[j]], add=True)
        return _

    lax.fori_loop(0, NBATCH, body, None)
    plsc.subcore_barrier()
    pltpu.sync_copy(accum.at[pl.ds(roff, RPT)],
                    out_hbm.at[cid, pl.ds(roff, RPT)])


@functools.partial(
    pl.kernel,
    out_type=jax.ShapeDtypeStruct((NC, NPAD, C), jnp.float32),
    mesh=_mesh,
    compiler_params=pltpu.CompilerParams(use_tc_tiling_on_sc=False,
                                         needs_layout_passes=False),
    scratch_types=[
        pltpu.VMEM((NBATCH, KB), jnp.int32),
        pltpu.VMEM((NBATCH, KB), jnp.int32),
        pltpu.VMEM((KB, H), jnp.float32),
        pltpu.VMEM((KB, H), jnp.float32),
        pltpu.VMEM((KB, C), jnp.float32),
        pltpu.VMEM((H, L), jnp.float32),
        pltpu.VMEM((L,), jnp.float32),
        pltpu.VMEM((ZR, C), jnp.float32),
        pltpu.VMEM_SHARED((NPAD, C), jnp.float32),
        pltpu.SemaphoreType.DMA,
        pltpu.SemaphoreType.DMA,
        pltpu.SemaphoreType.DMA,
    ],
)
def _sc4_edge_mlp(y_hbm, u_hbm, v_hbm, src_hbm, dst_hbm, w2b_hbm, b2b_hbm,
                  out_hbm,
                  src_v, dst_v, u_v, v_v, y_v, w2_v, b2_v, zbuf, accum,
                  sem_u, sem_v, sem_y):
    cid = lax.axis_index("c")
    sid = lax.axis_index("s")
    wid = sid * NC + cid
    roff = pl.multiple_of(sid * RPT, 8)

    _zero_accum_slice(zbuf, accum, roff)
    pltpu.sync_copy(src_hbm.at[wid], src_v)
    pltpu.sync_copy(dst_hbm.at[wid], dst_v)
    pltpu.sync_copy(w2b_hbm, w2_v)
    pltpu.sync_copy(b2b_hbm, b2_v)
    plsc.subcore_barrier()

    b2vec = b2_v[...]

    def body(j, _):
        cp_u = pltpu.async_copy(u_hbm.at[src_v.at[j]], u_v, sem_u)
        cp_v = pltpu.async_copy(v_hbm.at[dst_v.at[j]], v_v, sem_v)
        cp_y = pltpu.async_copy(y_hbm.at[src_v.at[j]], y_v, sem_y)
        cp_u.wait()
        cp_v.wait()
        cp_y.wait()

        for g in range(KB // L):
            eidx = lax.iota(jnp.int32, L) + g * L

            def kbody(kc, acc):
                kbase = kc * L
                for t in range(L):
                    k = kbase + t
                    kvec = jnp.full((L,), k, dtype=jnp.int32)
                    uk = plsc.load_gather(u_v, [eidx, kvec])
                    vk = plsc.load_gather(v_v, [eidx, kvec])
                    acc = acc + jnp.maximum(uk + vk, 0.0) * w2_v[k]
                return acc

            acc = lax.fori_loop(0, H // L, kbody,
                                jnp.zeros((L,), jnp.float32))
            ew = jnp.maximum(acc + b2vec, 0.0)
            for cc in range(C):
                cvec = jnp.full((L,), cc, dtype=jnp.int32)
                ycol = plsc.load_gather(y_v, [eidx, cvec])
                plsc.store_scatter(y_v, [eidx, cvec], ycol * ew)

        pltpu.sync_copy(y_v, accum.at[dst_v.at[j]], add=True)
        return _

    lax.fori_loop(0, NBATCH, body, None)
    plsc.subcore_barrier()
    pltpu.sync_copy(accum.at[pl.ds(roff, RPT)],
                    out_hbm.at[cid, pl.ds(roff, RPT)])


def kernel(x, edge_index, W_gcn, b_gcn, W1, b1, W2, b2):
    src = edge_index[0]
    dst = edge_index[1]
    pad = EPAD - E
    src_p = jnp.concatenate([src, jnp.zeros((pad,), jnp.int32)])
    dst_p = jnp.concatenate([dst, jnp.full((pad,), NPAD - 1, jnp.int32)])
    src3 = src_p.reshape(NW, NBATCH, KB)
    dst3 = dst_p.reshape(NW, NBATCH, KB)

    w1a = W1[:C]
    w1b = W1[C:]
    bg2 = b_gcn.reshape(1, C)
    b12 = b1.reshape(1, H)
    w2b = jnp.broadcast_to(W2.reshape(H, 1), (H, L))
    b2b = jnp.broadcast_to(b2.reshape(1), (L,))

    y = _tc1_y(x, W_gcn)
    part1 = _sc2_segsum(y, src3, dst3)
    u, v = _tc3_uv(part1[:, :N], bg2, w1a, w1b, b12)
    part2 = _sc4_edge_mlp(y, u, v, src3, dst3, w2b, b2b)
    return _tc5_out(part2[:, :N], bg2)
